# Initial kernel scaffold; baseline (speedup 1.0000x reference)
import jax
import jax.numpy as jnp
from jax.experimental import pallas as pl


def kernel(x, gcn_W0, gcn_b0, gcn_W1, gcn_b1, pai1_0, pai1_1, pai2_0, pai2_1, w1_W0, w1_b0, w1_W1, w1_b1, w2_W0, w2_b0, w2_W1, w2_b1, edge_index):
    raise NotImplementedError("write your pallas kernel here")


# speedup vs baseline: 20.9644x; 20.9644x over previous
import functools

import jax
import jax.numpy as jnp
from jax import lax
from jax.experimental import pallas as pl
from jax.experimental.pallas import tpu as pltpu, tpu_sc as plsc

N = 10000
DIN = 128
DH = 128
DOUT = 64
TH1 = 0.1
TH2 = 1.0

NC = 2
NS = 16
NW = NC * NS
NPAD = 10240
ZR = NPAD // NS

CH = 128
EPW = 10240
NCHK = EPW // CH
EP = NW * EPW

R = 1024
GRID = NPAD // R

_f32 = jnp.float32


def _act(v):
    w1 = (2.0 * TH2 - TH1) / TH2
    w2 = w1 - 1.0
    return (w1 * (jax.nn.relu(v - TH1) - jax.nn.relu(-v - TH1))
            - w2 * (jax.nn.relu(v - TH2) - jax.nn.relu(-v - TH2)))


@functools.cache
def _mesh():
    return plsc.VectorSubcoreMesh(core_axis_name="c", subcore_axis_name="s",
                                  num_cores=NC, num_subcores=NS)


def _fill(buf, n, val, idx=()):
    def body(i, _):
        buf[idx + (pl.ds(i * 16, 16),)] = jnp.full((16,), val, _f32)
        return 0

    lax.fori_loop(0, n // 16, body, 0, unroll=8)


def _make_degree():
    K = 4
    PH = NCHK // K

    @functools.partial(
        pl.kernel,
        out_type=jax.ShapeDtypeStruct((4 * NPAD,), _f32),
        mesh=_mesh(),
        scratch_types=[
            pltpu.VMEM((2, K, CH), jnp.int32),
            pltpu.VMEM((2, K, CH), jnp.int32),
            pltpu.VMEM((CH,), _f32),
            pltpu.VMEM((ZR,), _f32),
            pltpu.VMEM_SHARED((NPAD,), _f32),
            pltpu.VMEM_SHARED((NPAD,), _f32),
            pltpu.SemaphoreType.DMA,
            pltpu.SemaphoreType.DMA,
            pltpu.SemaphoreType.DMA,
            pltpu.SemaphoreType.DMA,
        ],
    )
    def degree(srcp, dstp, out_hbm, idxs, idxd, onesb, zb, acc_s, acc_d,
               isem0, isem1, ssem0, ssem1):
        c = lax.axis_index("c")
        s = lax.axis_index("s")
        w = c * NS + s
        isem = (isem0, isem1)
        ssem = (ssem0, ssem1)

        _fill(onesb, CH, 1.0)
        _fill(zb, ZR, 0.0)
        pltpu.sync_copy(zb, acc_s.at[pl.ds(s * ZR, ZR)])
        pltpu.sync_copy(zb, acc_d.at[pl.ds(s * ZR, ZR)])
        plsc.subcore_barrier()

        def fire_idx(p, x):
            pltpu.async_copy(srcp.at[w, pl.ds(p * K, K)], idxs.at[x], isem[x])
            pltpu.async_copy(dstp.at[w, pl.ds(p * K, K)], idxd.at[x], isem[x])

        def wait_idx(p, x):
            pltpu.make_async_copy(srcp.at[w, pl.ds(p * K, K)], idxs.at[x],
                                  isem[x]).wait()
            pltpu.make_async_copy(dstp.at[w, pl.ds(p * K, K)], idxd.at[x],
                                  isem[x]).wait()

        def drain_scatters(x):
            for k in range(K):
                pltpu.make_async_copy(onesb, acc_s.at[idxs.at[x, k]],
                                      ssem[x]).wait()
                pltpu.make_async_copy(onesb, acc_d.at[idxd.at[x, k]],
                                      ssem[x]).wait()

        def phase(p, x, first, last):
            y = 1 - x
            wait_idx(p, x)
            if first:
                pass
            else:
                drain_scatters(y)
            if last:
                pass
            else:
                @pl.when(p + 1 < PH)
                def _():
                    fire_idx(p + 1, y)
            for k in range(K):
                pltpu.async_copy(onesb, acc_s.at[idxs.at[x, k]], ssem[x],
                                 add=True)
                pltpu.async_copy(onesb, acc_d.at[idxd.at[x, k]], ssem[x],
                                 add=True)

        fire_idx(0, 0)

        def body(t, _):
            phase(2 * t, 0, False, False)
            phase(2 * t + 1, 1, False, False)
            return 0

        phase(0, 0, True, False)
        phase(1, 1, False, False)
        lax.fori_loop(1, PH // 2, body, 0)
        drain_scatters((PH - 1) % 2)
        plsc.subcore_barrier()
        pltpu.sync_copy(acc_s.at[pl.ds(s * ZR, ZR)],
                        out_hbm.at[pl.ds((2 * c) * NPAD + s * ZR, ZR)])
        pltpu.sync_copy(acc_d.at[pl.ds(s * ZR, ZR)],
                        out_hbm.at[pl.ds((2 * c + 1) * NPAD + s * ZR, ZR)])

    return degree


def _make_prop(d):
    K = 1 if d >= 128 else 4
    PH = NCHK // K
    NSL = 2 * K

    @functools.partial(
        pl.kernel,
        out_type=jax.ShapeDtypeStruct((2 * NPAD, d), _f32),
        mesh=_mesh(),
        compiler_params=pltpu.CompilerParams(
            use_tc_tiling_on_sc=(d % 128 == 0)),
        scratch_types=(
            [pltpu.VMEM((CH,), jnp.int32)] * NSL
            + [pltpu.VMEM((CH,), jnp.int32)] * NSL
            + [pltpu.VMEM((CH, d), _f32)] * NSL
            + [pltpu.VMEM_SHARED((NPAD, d), _f32)]
            + [pltpu.SemaphoreType.DMA] * 6
        ),
    )
    def prop(h_hbm, srcp, dstp, out_hbm, *sc):
        c = lax.axis_index("c")
        s = lax.axis_index("s")
        w = c * NS + s
        isl = [sc[x * K:(x + 1) * K] for x in range(2)]
        idl = [sc[NSL + x * K:NSL + (x + 1) * K] for x in range(2)]
        rl = [sc[2 * NSL + x * K:2 * NSL + (x + 1) * K] for x in range(2)]
        acc = sc[3 * NSL]
        isem = sc[3 * NSL + 1:3 * NSL + 3]
        gsem = sc[3 * NSL + 3:3 * NSL + 5]
        ssem = sc[3 * NSL + 5:3 * NSL + 7]

        zb = rl[0][0]

        def zrow(i, _):
            col = i % (d // 16)
            row = i // (d // 16)
            zb[row, pl.ds(col * 16, 16)] = jnp.zeros((16,), _f32)
            return 0

        lax.fori_loop(0, CH * d // 16, zrow, 0, unroll=8)
        for k in range(ZR // CH):
            pltpu.sync_copy(zb, acc.at[pl.ds(s * ZR + k * CH, CH)])
        plsc.subcore_barrier()

        def fire_idx(p, x):
            for k in range(K):
                pltpu.async_copy(srcp.at[w, p * K + k], isl[x][k], isem[x])
                pltpu.async_copy(dstp.at[w, p * K + k], idl[x][k], isem[x])

        def wait_idx(p, x):
            for k in range(K):
                pltpu.make_async_copy(srcp.at[w, p * K + k], isl[x][k],
                                      isem[x]).wait()
                pltpu.make_async_copy(dstp.at[w, p * K + k], idl[x][k],
                                      isem[x]).wait()

        def drain_scatters(x):
            for k in range(K):
                pltpu.make_async_copy(rl[x][k], acc.at[idl[x][k]],
                                      ssem[x]).wait()

        def phase(p, x, first):
            y = 1 - x
            wait_idx(p, x)
            for k in range(K):
                pltpu.async_copy(h_hbm.at[isl[x][k]], rl[x][k], gsem[x])
            if not first:
                drain_scatters(y)

            @pl.when(p + 1 < PH)
            def _():
                fire_idx(p + 1, y)

            for k in range(K):
                pltpu.make_async_copy(h_hbm.at[isl[x][k]], rl[x][k],
                                      gsem[x]).wait()
            for k in range(K):
                pltpu.async_copy(rl[x][k], acc.at[idl[x][k]], ssem[x],
                                 add=True)

        fire_idx(0, 0)
        phase(0, 0, True)
        phase(1, 1, False)

        def body(t, _):
            phase(2 * t, 0, False)
            phase(2 * t + 1, 1, False)
            return 0

        lax.fori_loop(1, PH // 2, body, 0)
        drain_scatters((PH - 1) % 2)
        plsc.subcore_barrier()
        pltpu.sync_copy(acc.at[pl.ds(s * ZR, ZR)],
                        out_hbm.at[pl.ds(c * NPAD + s * ZR, ZR)])

    return prop


def _cols_from_deg(deg):
    ds = deg[0:1] + deg[2:3]
    dd = deg[1:2] + deg[3:4]
    rs = lax.rsqrt(jnp.maximum(jnp.concatenate([ds, dd], axis=0), 1.0))
    eye2 = jnp.eye(2, dtype=_f32)
    cols = lax.dot_general(rs, eye2, (((0,), (0,)), ((), ())),
                           preferred_element_type=_f32)
    return cols[:, 0:1], cols[:, 1:2]


def _tc1_body(deg_ref, x_ref, w0_ref, ns_ref, nd_ref, h1_ref):
    ns, nd = _cols_from_deg(deg_ref[...])
    ns_ref[...] = ns
    nd_ref[...] = nd
    h1_ref[...] = jnp.dot(x_ref[...], w0_ref[...],
                          preferred_element_type=_f32) * ns


def _tc2_body(p_ref, ns_ref, nd_ref, b0_ref, w10_ref, b10_ref,
              ztp_ref, tp0_ref, g1_ref):
    prop = (p_ref[0] + p_ref[1]) * nd_ref[...]
    z = jax.nn.relu(prop + b0_ref[...])
    ztp_ref[...] = z
    tp0_ref[...] = jnp.tanh(jnp.dot(z, w10_ref[...],
                                    preferred_element_type=_f32) + b10_ref[...])
    g1_ref[...] = z * ns_ref[...]


def _tc3_body(ztp_ref, p_ref, x_ref, ns_ref, nd_ref, pai10_ref, pai20_ref,
              w20_ref, b20_ref, wg1_ref, emb0_ref, h2_ref):
    z = ztp_ref[...]
    propz = (p_ref[0] + p_ref[1]) * nd_ref[...]
    ze = (jnp.dot(z, pai10_ref[...], preferred_element_type=_f32)
          + jnp.dot(x_ref[...], pai20_ref[...], preferred_element_type=_f32)
          - z + propz)
    ze = _act(ze)
    emb0_ref[...] = jnp.tanh(jnp.dot(ze, w20_ref[...],
                                     preferred_element_type=_f32) + b20_ref[...])
    h2_ref[...] = jnp.dot(ze, wg1_ref[...],
                          preferred_element_type=_f32) * ns_ref[...]


def _tc4_body(p_ref, ns_ref, nd_ref, b1_ref, w11_ref, b11_ref,
              ztp_ref, tp1_ref, g2_ref):
    prop = (p_ref[0] + p_ref[1]) * nd_ref[...]
    z = jax.nn.relu(prop + b1_ref[...])
    ztp_ref[...] = z
    tp1_ref[...] = jnp.tanh(jnp.dot(z, w11_ref[...],
                                    preferred_element_type=_f32) + b11_ref[...])
    g2_ref[...] = z * ns_ref[...]


def _tc5_body(ztp_ref, p_ref, x_ref, nd_ref, pai11_ref, pai21_ref,
              w21_ref, b21_ref, emb1_ref):
    z = ztp_ref[...]
    propz = (p_ref[0] + p_ref[1]) * nd_ref[...]
    ze = (jnp.dot(z, pai11_ref[...], preferred_element_type=_f32)
          + jnp.dot(x_ref[...], pai21_ref[...], preferred_element_type=_f32)
          - z + propz)
    ze = _act(ze)
    emb1_ref[...] = jnp.tanh(jnp.dot(ze, w21_ref[...],
                                     preferred_element_type=_f32) + b21_ref[...])


def _rows(d):
    return pl.BlockSpec((R, d), lambda i: (i, 0))


def _part(d):
    return pl.BlockSpec((2, R, d), lambda i: (0, i, 0))


def _full(a, b):
    return pl.BlockSpec((a, b), lambda i: (0, 0))


_COL = pl.BlockSpec((R, 1), lambda i: (i, 0))
_DEG = pl.BlockSpec((4, R), lambda i: (0, i))


def _sd(shape):
    return jax.ShapeDtypeStruct(shape, _f32)


def kernel(x, gcn_W0, gcn_b0, gcn_W1, gcn_b1, pai1_0, pai1_1, pai2_0, pai2_1,
           w1_W0, w1_b0, w1_W1, w1_b1, w2_W0, w2_b0, w2_W1, w2_b1, edge_index):
    e = edge_index.shape[1]
    eidx = edge_index.astype(jnp.int32)
    npadedge = EP - e
    pad_idx = N + jnp.arange(npadedge, dtype=jnp.int32) % (NPAD - N)
    srcp = jnp.concatenate([eidx[0], pad_idx]).reshape(NW, NCHK, CH)
    dstp = jnp.concatenate([eidx[1], pad_idx]).reshape(NW, NCHK, CH)

    xp = jnp.pad(x, ((0, NPAD - N), (0, 0)))
    b0 = gcn_b0.reshape(1, DH)
    b1 = gcn_b1.reshape(1, DOUT)
    wb10 = w1_b0.reshape(1, DOUT)
    wb11 = w1_b1.reshape(1, DOUT)
    wb20 = w2_b0.reshape(1, DOUT)
    wb21 = w2_b1.reshape(1, DOUT)

    deg = _make_degree()(srcp, dstp).reshape(4, NPAD)

    ns, nd, h1 = pl.pallas_call(
        _tc1_body,
        grid=(GRID,),
        in_specs=[_DEG, _rows(DIN), _full(DIN, DH)],
        out_specs=[_COL, _COL, _rows(DH)],
        out_shape=[_sd((NPAD, 1)), _sd((NPAD, 1)), _sd((NPAD, DH))],
    )(deg, xp, gcn_W0)

    prop128 = _make_prop(DH)
    prop64 = _make_prop(DOUT)

    p1 = prop128(h1, srcp, dstp).reshape(2, NPAD, DH)

    ztp, tp0, g1 = pl.pallas_call(
        _tc2_body,
        grid=(GRID,),
        in_specs=[_part(DH), _COL, _COL, _full(1, DH), _full(DH, DOUT),
                  _full(1, DOUT)],
        out_specs=[_rows(DH), _rows(DOUT), _rows(DH)],
        out_shape=[_sd((NPAD, DH)), _sd((NPAD, DOUT)), _sd((NPAD, DH))],
    )(p1, ns, nd, b0, w1_W0, wb10)

    p2 = prop128(g1, srcp, dstp).reshape(2, NPAD, DH)

    emb0, h2 = pl.pallas_call(
        _tc3_body,
        grid=(GRID,),
        in_specs=[_rows(DH), _part(DH), _rows(DIN), _COL, _COL,
                  _full(DH, DH), _full(DIN, DH), _full(DH, DOUT),
                  _full(1, DOUT), _full(DH, DOUT)],
        out_specs=[_rows(DOUT), _rows(DOUT)],
        out_shape=[_sd((NPAD, DOUT)), _sd((NPAD, DOUT))],
    )(ztp, p2, xp, ns, nd, pai1_0, pai2_0, w2_W0, wb20, gcn_W1)

    p3 = prop64(h2, srcp, dstp).reshape(2, NPAD, DOUT)

    ztp2, tp1, g2 = pl.pallas_call(
        _tc4_body,
        grid=(GRID,),
        in_specs=[_part(DOUT), _COL, _COL, _full(1, DOUT), _full(DOUT, DOUT),
                  _full(1, DOUT)],
        out_specs=[_rows(DOUT), _rows(DOUT), _rows(DOUT)],
        out_shape=[_sd((NPAD, DOUT)), _sd((NPAD, DOUT)), _sd((NPAD, DOUT))],
    )(p3, ns, nd, b1, w1_W1, wb11)

    p4 = prop64(g2, srcp, dstp).reshape(2, NPAD, DOUT)

    (emb1,) = pl.pallas_call(
        _tc5_body,
        grid=(GRID,),
        in_specs=[_rows(DOUT), _part(DOUT), _rows(DIN), _COL,
                  _full(DOUT, DOUT), _full(DIN, DOUT), _full(DOUT, DOUT),
                  _full(1, DOUT)],
        out_specs=[_rows(DOUT)],
        out_shape=[_sd((NPAD, DOUT))],
    )(ztp2, p4, xp, nd, pai1_1, pai2_1, w2_W1, wb21)

    return (tp0[:N], emb0[:N], tp1[:N], emb1[:N])


# speedup vs baseline: 21.1291x; 1.0079x over previous
import functools

import jax
import jax.numpy as jnp
from jax import lax
from jax.experimental import pallas as pl
from jax.experimental.pallas import tpu as pltpu, tpu_sc as plsc

N = 10000
DIN = 128
DH = 128
DOUT = 64
TH1 = 0.1
TH2 = 1.0

NC = 2
NS = 16
NW = NC * NS
NPAD = 10240
ZR = NPAD // NS

CH = 128
EPW = 10240
NCHK = EPW // CH
EP = NW * EPW

R = 1024
GRID = NPAD // R

_f32 = jnp.float32


def _act(v):
    w1 = (2.0 * TH2 - TH1) / TH2
    w2 = w1 - 1.0
    return (w1 * (jax.nn.relu(v - TH1) - jax.nn.relu(-v - TH1))
            - w2 * (jax.nn.relu(v - TH2) - jax.nn.relu(-v - TH2)))


@functools.cache
def _mesh():
    return plsc.VectorSubcoreMesh(core_axis_name="c", subcore_axis_name="s",
                                  num_cores=NC, num_subcores=NS)


def _fill(buf, n, val, idx=()):
    def body(i, _):
        buf[idx + (pl.ds(i * 16, 16),)] = jnp.full((16,), val, _f32)
        return 0

    lax.fori_loop(0, n // 16, body, 0, unroll=8)


def _make_degree():
    K = 4
    PH = NCHK // K

    @functools.partial(
        pl.kernel,
        out_type=jax.ShapeDtypeStruct((4 * NPAD,), _f32),
        mesh=_mesh(),
        scratch_types=[
            pltpu.VMEM((2, K, CH), jnp.int32),
            pltpu.VMEM((2, K, CH), jnp.int32),
            pltpu.VMEM((CH,), _f32),
            pltpu.VMEM((ZR,), _f32),
            pltpu.VMEM_SHARED((NPAD,), _f32),
            pltpu.VMEM_SHARED((NPAD,), _f32),
            pltpu.SemaphoreType.DMA,
            pltpu.SemaphoreType.DMA,
            pltpu.SemaphoreType.DMA,
            pltpu.SemaphoreType.DMA,
        ],
    )
    def degree(srcp, dstp, out_hbm, idxs, idxd, onesb, zb, acc_s, acc_d,
               isem0, isem1, ssem0, ssem1):
        c = lax.axis_index("c")
        s = lax.axis_index("s")
        w = c * NS + s
        isem = (isem0, isem1)
        ssem = (ssem0, ssem1)

        _fill(onesb, CH, 1.0)
        _fill(zb, ZR, 0.0)
        pltpu.sync_copy(zb, acc_s.at[pl.ds(s * ZR, ZR)])
        pltpu.sync_copy(zb, acc_d.at[pl.ds(s * ZR, ZR)])
        plsc.subcore_barrier()

        def fire_idx(p, x):
            pltpu.async_copy(srcp.at[w, pl.ds(p * K, K)], idxs.at[x], isem[x])
            pltpu.async_copy(dstp.at[w, pl.ds(p * K, K)], idxd.at[x], isem[x])

        def wait_idx(p, x):
            pltpu.make_async_copy(srcp.at[w, pl.ds(p * K, K)], idxs.at[x],
                                  isem[x]).wait()
            pltpu.make_async_copy(dstp.at[w, pl.ds(p * K, K)], idxd.at[x],
                                  isem[x]).wait()

        def drain_scatters(x):
            for k in range(K):
                pltpu.make_async_copy(onesb, acc_s.at[idxs.at[x, k]],
                                      ssem[x]).wait()
                pltpu.make_async_copy(onesb, acc_d.at[idxd.at[x, k]],
                                      ssem[x]).wait()

        def phase(p, x, first, last):
            y = 1 - x
            wait_idx(p, x)
            if first:
                pass
            else:
                drain_scatters(y)
            if last:
                pass
            else:
                @pl.when(p + 1 < PH)
                def _():
                    fire_idx(p + 1, y)
            for k in range(K):
                pltpu.async_copy(onesb, acc_s.at[idxs.at[x, k]], ssem[x],
                                 add=True)
                pltpu.async_copy(onesb, acc_d.at[idxd.at[x, k]], ssem[x],
                                 add=True)

        fire_idx(0, 0)

        def body(t, _):
            phase(2 * t, 0, False, False)
            phase(2 * t + 1, 1, False, False)
            return 0

        phase(0, 0, True, False)
        phase(1, 1, False, False)
        lax.fori_loop(1, PH // 2, body, 0)
        drain_scatters((PH - 1) % 2)
        plsc.subcore_barrier()
        pltpu.sync_copy(acc_s.at[pl.ds(s * ZR, ZR)],
                        out_hbm.at[pl.ds((2 * c) * NPAD + s * ZR, ZR)])
        pltpu.sync_copy(acc_d.at[pl.ds(s * ZR, ZR)],
                        out_hbm.at[pl.ds((2 * c + 1) * NPAD + s * ZR, ZR)])

    return degree


def _make_prop(d):
    K = 1 if d >= 128 else 4
    PH = NCHK // K
    NSL = 2 * K
    PRELOAD_SRC = d < 128

    src_slots = [] if PRELOAD_SRC else [pltpu.VMEM((CH,), jnp.int32)] * 2
    src_pre = [pltpu.VMEM((NCHK, CH), jnp.int32)] if PRELOAD_SRC else []

    @functools.partial(
        pl.kernel,
        out_type=jax.ShapeDtypeStruct((2 * NPAD, d), _f32),
        mesh=_mesh(),
        compiler_params=pltpu.CompilerParams(
            use_tc_tiling_on_sc=(d % 128 == 0)),
        scratch_types=(
            src_slots + src_pre
            + [pltpu.VMEM((NCHK, CH), jnp.int32)]
            + [pltpu.VMEM((CH, d), _f32)] * NSL
            + [pltpu.VMEM_SHARED((NPAD, d), _f32)]
            + [pltpu.SemaphoreType.DMA] * 5
        ),
    )
    def prop(h_hbm, srcp, dstp, out_hbm, *sc):
        c = lax.axis_index("c")
        s = lax.axis_index("s")
        w = c * NS + s
        if PRELOAD_SRC:
            srcall = sc[0]
            nfix = 1
        else:
            ssl = sc[0:2]
            nfix = 2
        dstall = sc[nfix]
        rl = [sc[nfix + 1 + x * K:nfix + 1 + (x + 1) * K] for x in range(2)]
        acc = sc[nfix + 1 + NSL]
        isem = sc[nfix + 2 + NSL:nfix + 4 + NSL]
        gsem = sc[nfix + 4 + NSL]
        ssem = sc[nfix + 5 + NSL:nfix + 7 + NSL]

        pltpu.async_copy(dstp.at[w], dstall, isem[0])
        if PRELOAD_SRC:
            pltpu.async_copy(srcp.at[w], srcall, isem[1])
        zb = rl[0][0]

        def zrow(i, _):
            col = i % (d // 16)
            row = i // (d // 16)
            zb[row, pl.ds(col * 16, 16)] = jnp.zeros((16,), _f32)
            return 0

        lax.fori_loop(0, CH * d // 16, zrow, 0, unroll=8)
        for k in range(ZR // CH):
            pltpu.sync_copy(zb, acc.at[pl.ds(s * ZR + k * CH, CH)])
        pltpu.make_async_copy(dstp.at[w], dstall, isem[0]).wait()
        if PRELOAD_SRC:
            pltpu.make_async_copy(srcp.at[w], srcall, isem[1]).wait()
        plsc.subcore_barrier()

        def sidx(p, k, x):
            if PRELOAD_SRC:
                return srcall.at[p * K + k]
            return ssl[x]

        def drain_scatters(p, x):
            for k in range(K):
                pltpu.make_async_copy(rl[x][k], acc.at[dstall.at[p * K + k]],
                                      ssem[x]).wait()

        def phase(p, x, first):
            y = 1 - x
            if not PRELOAD_SRC:
                pltpu.make_async_copy(srcp.at[w, p], ssl[x], isem[x]).wait()
            for k in range(K):
                pltpu.async_copy(h_hbm.at[sidx(p, k, x)], rl[x][k], gsem)
            if not first:
                drain_scatters(p - 1, y)
            if not PRELOAD_SRC:
                @pl.when(p + 1 < PH)
                def _():
                    pltpu.async_copy(srcp.at[w, p + 1], ssl[y], isem[y])
            for k in range(K):
                pltpu.make_async_copy(h_hbm.at[sidx(p, k, x)], rl[x][k],
                                      gsem).wait()
            for k in range(K):
                pltpu.async_copy(rl[x][k], acc.at[dstall.at[p * K + k]],
                                 ssem[x], add=True)

        if not PRELOAD_SRC:
            pltpu.async_copy(srcp.at[w, 0], ssl[0], isem[0])
        phase(0, 0, True)
        phase(1, 1, False)

        def body(t, _):
            phase(2 * t, 0, False)
            phase(2 * t + 1, 1, False)
            return 0

        lax.fori_loop(1, PH // 2, body, 0)
        drain_scatters(PH - 1, (PH - 1) % 2)
        plsc.subcore_barrier()
        pltpu.sync_copy(acc.at[pl.ds(s * ZR, ZR)],
                        out_hbm.at[pl.ds(c * NPAD + s * ZR, ZR)])

    return prop


def _cols_from_deg(deg):
    ds = deg[0:1] + deg[2:3]
    dd = deg[1:2] + deg[3:4]
    rs = lax.rsqrt(jnp.maximum(jnp.concatenate([ds, dd], axis=0), 1.0))
    eye2 = jnp.eye(2, dtype=_f32)
    cols = lax.dot_general(rs, eye2, (((0,), (0,)), ((), ())),
                           preferred_element_type=_f32)
    return cols[:, 0:1], cols[:, 1:2]


def _tc1_body(deg_ref, x_ref, w0_ref, ns_ref, nd_ref, h1_ref):
    ns, nd = _cols_from_deg(deg_ref[...])
    ns_ref[...] = ns
    nd_ref[...] = nd
    h1_ref[...] = jnp.dot(x_ref[...], w0_ref[...],
                          preferred_element_type=_f32) * ns


def _tc2_body(pa_ref, pb_ref, ns_ref, nd_ref, b0_ref, w10_ref, b10_ref,
              ztp_ref, tp0_ref, g1_ref):
    prop = (pa_ref[...] + pb_ref[...]) * nd_ref[...]
    z = jax.nn.relu(prop + b0_ref[...])
    ztp_ref[...] = z
    tp0_ref[...] = jnp.tanh(jnp.dot(z, w10_ref[...],
                                    preferred_element_type=_f32) + b10_ref[...])
    g1_ref[...] = z * ns_ref[...]


def _tc3_body(ztp_ref, pa_ref, pb_ref, x_ref, ns_ref, nd_ref, pai10_ref,
              pai20_ref, w20_ref, b20_ref, wg1_ref, emb0_ref, h2_ref):
    z = ztp_ref[...]
    propz = (pa_ref[...] + pb_ref[...]) * nd_ref[...]
    ze = (jnp.dot(z, pai10_ref[...], preferred_element_type=_f32)
          + jnp.dot(x_ref[...], pai20_ref[...], preferred_element_type=_f32)
          - z + propz)
    ze = _act(ze)
    emb0_ref[...] = jnp.tanh(jnp.dot(ze, w20_ref[...],
                                     preferred_element_type=_f32) + b20_ref[...])
    h2_ref[...] = jnp.dot(ze, wg1_ref[...],
                          preferred_element_type=_f32) * ns_ref[...]


def _tc4_body(pa_ref, pb_ref, ns_ref, nd_ref, b1_ref, w11_ref, b11_ref,
              ztp_ref, tp1_ref, g2_ref):
    prop = (pa_ref[...] + pb_ref[...]) * nd_ref[...]
    z = jax.nn.relu(prop + b1_ref[...])
    ztp_ref[...] = z
    tp1_ref[...] = jnp.tanh(jnp.dot(z, w11_ref[...],
                                    preferred_element_type=_f32) + b11_ref[...])
    g2_ref[...] = z * ns_ref[...]


def _tc5_body(ztp_ref, pa_ref, pb_ref, x_ref, nd_ref, pai11_ref, pai21_ref,
              w21_ref, b21_ref, emb1_ref):
    z = ztp_ref[...]
    propz = (pa_ref[...] + pb_ref[...]) * nd_ref[...]
    ze = (jnp.dot(z, pai11_ref[...], preferred_element_type=_f32)
          + jnp.dot(x_ref[...], pai21_ref[...], preferred_element_type=_f32)
          - z + propz)
    ze = _act(ze)
    emb1_ref[...] = jnp.tanh(jnp.dot(ze, w21_ref[...],
                                     preferred_element_type=_f32) + b21_ref[...])


def _rows(d):
    return pl.BlockSpec((R, d), lambda i: (i, 0))


def _parta(d):
    return pl.BlockSpec((R, d), lambda i: (i, 0))


def _partb(d):
    return pl.BlockSpec((R, d), lambda i: (i + GRID, 0))


def _full(a, b):
    return pl.BlockSpec((a, b), lambda i: (0, 0))


_COL = pl.BlockSpec((R, 1), lambda i: (i, 0))
_DEG = pl.BlockSpec((4, R), lambda i: (0, i))


def _sd(shape):
    return jax.ShapeDtypeStruct(shape, _f32)


def kernel(x, gcn_W0, gcn_b0, gcn_W1, gcn_b1, pai1_0, pai1_1, pai2_0, pai2_1,
           w1_W0, w1_b0, w1_W1, w1_b1, w2_W0, w2_b0, w2_W1, w2_b1, edge_index):
    e = edge_index.shape[1]
    eidx = edge_index.astype(jnp.int32)
    npadedge = EP - e
    pad_idx = N + jnp.arange(npadedge, dtype=jnp.int32) % (NPAD - N)
    srcp = jnp.concatenate([eidx[0], pad_idx]).reshape(NW, NCHK, CH)
    dstp = jnp.concatenate([eidx[1], pad_idx]).reshape(NW, NCHK, CH)

    xp = jnp.pad(x, ((0, NPAD - N), (0, 0)))
    b0 = gcn_b0.reshape(1, DH)
    b1 = gcn_b1.reshape(1, DOUT)
    wb10 = w1_b0.reshape(1, DOUT)
    wb11 = w1_b1.reshape(1, DOUT)
    wb20 = w2_b0.reshape(1, DOUT)
    wb21 = w2_b1.reshape(1, DOUT)

    deg = _make_degree()(srcp, dstp).reshape(4, NPAD)

    ns, nd, h1 = pl.pallas_call(
        _tc1_body,
        grid=(GRID,),
        in_specs=[_DEG, _rows(DIN), _full(DIN, DH)],
        out_specs=[_COL, _COL, _rows(DH)],
        out_shape=[_sd((NPAD, 1)), _sd((NPAD, 1)), _sd((NPAD, DH))],
    )(deg, xp, gcn_W0)

    prop128 = _make_prop(DH)
    prop64 = _make_prop(DOUT)

    p1 = prop128(h1, srcp, dstp)

    ztp, tp0, g1 = pl.pallas_call(
        _tc2_body,
        grid=(GRID,),
        in_specs=[_parta(DH), _partb(DH), _COL, _COL, _full(1, DH),
                  _full(DH, DOUT), _full(1, DOUT)],
        out_specs=[_rows(DH), _rows(DOUT), _rows(DH)],
        out_shape=[_sd((NPAD, DH)), _sd((N, DOUT)), _sd((NPAD, DH))],
    )(p1, p1, ns, nd, b0, w1_W0, wb10)

    p2 = prop128(g1, srcp, dstp)

    emb0, h2 = pl.pallas_call(
        _tc3_body,
        grid=(GRID,),
        in_specs=[_rows(DH), _parta(DH), _partb(DH), _rows(DIN), _COL, _COL,
                  _full(DH, DH), _full(DIN, DH), _full(DH, DOUT),
                  _full(1, DOUT), _full(DH, DOUT)],
        out_specs=[_rows(DOUT), _rows(DOUT)],
        out_shape=[_sd((N, DOUT)), _sd((NPAD, DOUT))],
    )(ztp, p2, p2, xp, ns, nd, pai1_0, pai2_0, w2_W0, wb20, gcn_W1)

    p3 = prop64(h2, srcp, dstp)

    ztp2, tp1, g2 = pl.pallas_call(
        _tc4_body,
        grid=(GRID,),
        in_specs=[_parta(DOUT), _partb(DOUT), _COL, _COL, _full(1, DOUT),
                  _full(DOUT, DOUT), _full(1, DOUT)],
        out_specs=[_rows(DOUT), _rows(DOUT), _rows(DOUT)],
        out_shape=[_sd((NPAD, DOUT)), _sd((N, DOUT)), _sd((NPAD, DOUT))],
    )(p3, p3, ns, nd, b1, w1_W1, wb11)

    p4 = prop64(g2, srcp, dstp)

    (emb1,) = pl.pallas_call(
        _tc5_body,
        grid=(GRID,),
        in_specs=[_rows(DOUT), _parta(DOUT), _partb(DOUT), _rows(DIN), _COL,
                  _full(DOUT, DOUT), _full(DIN, DOUT), _full(DOUT, DOUT),
                  _full(1, DOUT)],
        out_specs=[_rows(DOUT)],
        out_shape=[_sd((N, DOUT))],
    )(ztp2, p4, p4, xp, nd, pai1_1, pai2_1, w2_W1, wb21)

    return (tp0, emb0, tp1, emb1)


# speedup vs baseline: 21.4745x; 1.0164x over previous
import functools

import jax
import jax.numpy as jnp
from jax import lax
from jax.experimental import pallas as pl
from jax.experimental.pallas import tpu as pltpu, tpu_sc as plsc

N = 10000
DIN = 128
DH = 128
DOUT = 64
TH1 = 0.1
TH2 = 1.0

NC = 2
NS = 16
NW = NC * NS
NPAD = 10240
ZR = NPAD // NS

CH = 128
EPW = 10240
NCHK = EPW // CH
EP = NW * EPW

R = 2048
GRID = NPAD // R

_f32 = jnp.float32


def _act(v):
    w1 = (2.0 * TH2 - TH1) / TH2
    w2 = w1 - 1.0
    return (w1 * (jax.nn.relu(v - TH1) - jax.nn.relu(-v - TH1))
            - w2 * (jax.nn.relu(v - TH2) - jax.nn.relu(-v - TH2)))


@functools.cache
def _mesh():
    return plsc.VectorSubcoreMesh(core_axis_name="c", subcore_axis_name="s",
                                  num_cores=NC, num_subcores=NS)


def _fill(buf, n, val, idx=()):
    def body(i, _):
        buf[idx + (pl.ds(i * 16, 16),)] = jnp.full((16,), val, _f32)
        return 0

    lax.fori_loop(0, n // 16, body, 0, unroll=8)


def _make_degree():
    K = 4
    PH = NCHK // K

    @functools.partial(
        pl.kernel,
        out_type=jax.ShapeDtypeStruct((4 * NPAD,), _f32),
        mesh=_mesh(),
        scratch_types=[
            pltpu.VMEM((2, K, CH), jnp.int32),
            pltpu.VMEM((2, K, CH), jnp.int32),
            pltpu.VMEM((CH,), _f32),
            pltpu.VMEM((ZR,), _f32),
            pltpu.VMEM_SHARED((NPAD,), _f32),
            pltpu.VMEM_SHARED((NPAD,), _f32),
            pltpu.SemaphoreType.DMA,
            pltpu.SemaphoreType.DMA,
            pltpu.SemaphoreType.DMA,
            pltpu.SemaphoreType.DMA,
        ],
    )
    def degree(srcp, dstp, out_hbm, idxs, idxd, onesb, zb, acc_s, acc_d,
               isem0, isem1, ssem0, ssem1):
        c = lax.axis_index("c")
        s = lax.axis_index("s")
        w = c * NS + s
        isem = (isem0, isem1)
        ssem = (ssem0, ssem1)

        _fill(onesb, CH, 1.0)
        _fill(zb, ZR, 0.0)
        pltpu.sync_copy(zb, acc_s.at[pl.ds(s * ZR, ZR)])
        pltpu.sync_copy(zb, acc_d.at[pl.ds(s * ZR, ZR)])
        plsc.subcore_barrier()

        def fire_idx(p, x):
            pltpu.async_copy(srcp.at[w, pl.ds(p * K, K)], idxs.at[x], isem[x])
            pltpu.async_copy(dstp.at[w, pl.ds(p * K, K)], idxd.at[x], isem[x])

        def wait_idx(p, x):
            pltpu.make_async_copy(srcp.at[w, pl.ds(p * K, K)], idxs.at[x],
                                  isem[x]).wait()
            pltpu.make_async_copy(dstp.at[w, pl.ds(p * K, K)], idxd.at[x],
                                  isem[x]).wait()

        def drain_scatters(x):
            for k in range(K):
                pltpu.make_async_copy(onesb, acc_s.at[idxs.at[x, k]],
                                      ssem[x]).wait()
                pltpu.make_async_copy(onesb, acc_d.at[idxd.at[x, k]],
                                      ssem[x]).wait()

        def phase(p, x, first, last):
            y = 1 - x
            wait_idx(p, x)
            if first:
                pass
            else:
                drain_scatters(y)
            if last:
                pass
            else:
                @pl.when(p + 1 < PH)
                def _():
                    fire_idx(p + 1, y)
            for k in range(K):
                pltpu.async_copy(onesb, acc_s.at[idxs.at[x, k]], ssem[x],
                                 add=True)
                pltpu.async_copy(onesb, acc_d.at[idxd.at[x, k]], ssem[x],
                                 add=True)

        fire_idx(0, 0)

        def body(t, _):
            phase(2 * t, 0, False, False)
            phase(2 * t + 1, 1, False, False)
            return 0

        phase(0, 0, True, False)
        phase(1, 1, False, False)
        lax.fori_loop(1, PH // 2, body, 0)
        drain_scatters((PH - 1) % 2)
        plsc.subcore_barrier()
        pltpu.sync_copy(acc_s.at[pl.ds(s * ZR, ZR)],
                        out_hbm.at[pl.ds((2 * c) * NPAD + s * ZR, ZR)])
        pltpu.sync_copy(acc_d.at[pl.ds(s * ZR, ZR)],
                        out_hbm.at[pl.ds((2 * c + 1) * NPAD + s * ZR, ZR)])

    return degree


def _make_prop(d):
    K = 1 if d >= 128 else 4
    PH = NCHK // K
    NSL = 2 * K
    PRELOAD_SRC = d < 128

    src_slots = [] if PRELOAD_SRC else [pltpu.VMEM((CH,), jnp.int32)] * 2
    src_pre = [pltpu.VMEM((NCHK, CH), jnp.int32)] if PRELOAD_SRC else []

    @functools.partial(
        pl.kernel,
        out_type=jax.ShapeDtypeStruct((2 * NPAD, d), _f32),
        mesh=_mesh(),
        compiler_params=pltpu.CompilerParams(
            use_tc_tiling_on_sc=(d % 128 == 0)),
        scratch_types=(
            src_slots + src_pre
            + [pltpu.VMEM((NCHK, CH), jnp.int32)]
            + [pltpu.VMEM((CH, d), _f32)] * NSL
            + [pltpu.VMEM_SHARED((NPAD, d), _f32)]
            + [pltpu.SemaphoreType.DMA] * 5
        ),
    )
    def prop(h_hbm, srcp, dstp, out_hbm, *sc):
        c = lax.axis_index("c")
        s = lax.axis_index("s")
        w = c * NS + s
        if PRELOAD_SRC:
            srcall = sc[0]
            nfix = 1
        else:
            ssl = sc[0:2]
            nfix = 2
        dstall = sc[nfix]
        rl = [sc[nfix + 1 + x * K:nfix + 1 + (x + 1) * K] for x in range(2)]
        acc = sc[nfix + 1 + NSL]
        isem = sc[nfix + 2 + NSL:nfix + 4 + NSL]
        gsem = sc[nfix + 4 + NSL]
        ssem = sc[nfix + 5 + NSL:nfix + 7 + NSL]

        pltpu.async_copy(dstp.at[w], dstall, isem[0])
        if PRELOAD_SRC:
            pltpu.async_copy(srcp.at[w], srcall, isem[1])
        zb = rl[0][0]

        def zrow(i, _):
            col = i % (d // 16)
            row = i // (d // 16)
            zb[row, pl.ds(col * 16, 16)] = jnp.zeros((16,), _f32)
            return 0

        lax.fori_loop(0, CH * d // 16, zrow, 0, unroll=8)
        for k in range(ZR // CH):
            pltpu.sync_copy(zb, acc.at[pl.ds(s * ZR + k * CH, CH)])
        pltpu.make_async_copy(dstp.at[w], dstall, isem[0]).wait()
        if PRELOAD_SRC:
            pltpu.make_async_copy(srcp.at[w], srcall, isem[1]).wait()
        plsc.subcore_barrier()

        def sidx(p, k, x):
            if PRELOAD_SRC:
                return srcall.at[p * K + k]
            return ssl[x]

        def drain_scatters(p, x):
            for k in range(K):
                pltpu.make_async_copy(rl[x][k], acc.at[dstall.at[p * K + k]],
                                      ssem[x]).wait()

        def phase(p, x, first):
            y = 1 - x
            if not PRELOAD_SRC:
                pltpu.make_async_copy(srcp.at[w, p], ssl[x], isem[x]).wait()
            for k in range(K):
                pltpu.async_copy(h_hbm.at[sidx(p, k, x)], rl[x][k], gsem)
            if not first:
                drain_scatters(p - 1, y)
            if not PRELOAD_SRC:
                @pl.when(p + 1 < PH)
                def _():
                    pltpu.async_copy(srcp.at[w, p + 1], ssl[y], isem[y])
            for k in range(K):
                pltpu.make_async_copy(h_hbm.at[sidx(p, k, x)], rl[x][k],
                                      gsem).wait()
            for k in range(K):
                pltpu.async_copy(rl[x][k], acc.at[dstall.at[p * K + k]],
                                 ssem[x], add=True)

        if not PRELOAD_SRC:
            pltpu.async_copy(srcp.at[w, 0], ssl[0], isem[0])
        phase(0, 0, True)
        phase(1, 1, False)

        def body(t, _):
            phase(2 * t, 0, False)
            phase(2 * t + 1, 1, False)
            return 0

        lax.fori_loop(1, PH // 2, body, 0)
        drain_scatters(PH - 1, (PH - 1) % 2)
        plsc.subcore_barrier()
        pltpu.sync_copy(acc.at[pl.ds(s * ZR, ZR)],
                        out_hbm.at[pl.ds(c * NPAD + s * ZR, ZR)])

    return prop


def _cols_from_deg(deg):
    ds = deg[0:1] + deg[2:3]
    dd = deg[1:2] + deg[3:4]
    rs = lax.rsqrt(jnp.maximum(jnp.concatenate([ds, dd], axis=0), 1.0))
    eye2 = jnp.eye(2, dtype=_f32)
    cols = lax.dot_general(rs, eye2, (((0,), (0,)), ((), ())),
                           preferred_element_type=_f32)
    return cols[:, 0:1], cols[:, 1:2]


def _tc1_body(deg_ref, x_ref, w0_ref, ns_ref, nd_ref, h1_ref):
    ns, nd = _cols_from_deg(deg_ref[...])
    ns_ref[...] = ns
    nd_ref[...] = nd
    h1_ref[...] = jnp.dot(x_ref[...], w0_ref[...],
                          preferred_element_type=_f32) * ns


def _tc2_body(pa_ref, pb_ref, ns_ref, nd_ref, b0_ref, w10_ref, b10_ref,
              ztp_ref, tp0_ref, g1_ref):
    prop = (pa_ref[...] + pb_ref[...]) * nd_ref[...]
    z = jax.nn.relu(prop + b0_ref[...])
    ztp_ref[...] = z
    tp0_ref[...] = jnp.tanh(jnp.dot(z, w10_ref[...],
                                    preferred_element_type=_f32) + b10_ref[...])
    g1_ref[...] = z * ns_ref[...]


def _tc3_body(ztp_ref, pa_ref, pb_ref, x_ref, ns_ref, nd_ref, pai10_ref,
              pai20_ref, w20_ref, b20_ref, wg1_ref, emb0_ref, h2_ref):
    z = ztp_ref[...]
    propz = (pa_ref[...] + pb_ref[...]) * nd_ref[...]
    ze = (jnp.dot(z, pai10_ref[...], preferred_element_type=_f32)
          + jnp.dot(x_ref[...], pai20_ref[...], preferred_element_type=_f32)
          - z + propz)
    ze = _act(ze)
    emb0_ref[...] = jnp.tanh(jnp.dot(ze, w20_ref[...],
                                     preferred_element_type=_f32) + b20_ref[...])
    h2_ref[...] = jnp.dot(ze, wg1_ref[...],
                          preferred_element_type=_f32) * ns_ref[...]


def _tc4_body(pa_ref, pb_ref, ns_ref, nd_ref, b1_ref, w11_ref, b11_ref,
              ztp_ref, tp1_ref, g2_ref):
    prop = (pa_ref[...] + pb_ref[...]) * nd_ref[...]
    z = jax.nn.relu(prop + b1_ref[...])
    ztp_ref[...] = z
    tp1_ref[...] = jnp.tanh(jnp.dot(z, w11_ref[...],
                                    preferred_element_type=_f32) + b11_ref[...])
    g2_ref[...] = z * ns_ref[...]


def _tc5_body(ztp_ref, pa_ref, pb_ref, x_ref, nd_ref, pai11_ref, pai21_ref,
              w21_ref, b21_ref, emb1_ref):
    z = ztp_ref[...]
    propz = (pa_ref[...] + pb_ref[...]) * nd_ref[...]
    ze = (jnp.dot(z, pai11_ref[...], preferred_element_type=_f32)
          + jnp.dot(x_ref[...], pai21_ref[...], preferred_element_type=_f32)
          - z + propz)
    ze = _act(ze)
    emb1_ref[...] = jnp.tanh(jnp.dot(ze, w21_ref[...],
                                     preferred_element_type=_f32) + b21_ref[...])


def _rows(d):
    return pl.BlockSpec((R, d), lambda i: (i, 0))


def _parta(d):
    return pl.BlockSpec((R, d), lambda i: (i, 0))


def _partb(d):
    return pl.BlockSpec((R, d), lambda i: (i + GRID, 0))


def _full(a, b):
    return pl.BlockSpec((a, b), lambda i: (0, 0))


_COL = pl.BlockSpec((R, 1), lambda i: (i, 0))
_DEG = pl.BlockSpec((4, R), lambda i: (0, i))


def _sd(shape):
    return jax.ShapeDtypeStruct(shape, _f32)


def kernel(x, gcn_W0, gcn_b0, gcn_W1, gcn_b1, pai1_0, pai1_1, pai2_0, pai2_1,
           w1_W0, w1_b0, w1_W1, w1_b1, w2_W0, w2_b0, w2_W1, w2_b1, edge_index):
    e = edge_index.shape[1]
    eidx = edge_index.astype(jnp.int32)
    npadedge = EP - e
    pad_idx = N + jnp.arange(npadedge, dtype=jnp.int32) % (NPAD - N)
    srcp = jnp.concatenate([eidx[0], pad_idx]).reshape(NW, NCHK, CH)
    dstp = jnp.concatenate([eidx[1], pad_idx]).reshape(NW, NCHK, CH)

    xp = jnp.pad(x, ((0, NPAD - N), (0, 0)))
    b0 = gcn_b0.reshape(1, DH)
    b1 = gcn_b1.reshape(1, DOUT)
    wb10 = w1_b0.reshape(1, DOUT)
    wb11 = w1_b1.reshape(1, DOUT)
    wb20 = w2_b0.reshape(1, DOUT)
    wb21 = w2_b1.reshape(1, DOUT)

    deg = _make_degree()(srcp, dstp).reshape(4, NPAD)

    ns, nd, h1 = pl.pallas_call(
        _tc1_body,
        grid=(GRID,),
        in_specs=[_DEG, _rows(DIN), _full(DIN, DH)],
        out_specs=[_COL, _COL, _rows(DH)],
        out_shape=[_sd((NPAD, 1)), _sd((NPAD, 1)), _sd((NPAD, DH))],
    )(deg, xp, gcn_W0)

    prop128 = _make_prop(DH)
    prop64 = _make_prop(DOUT)

    p1 = prop128(h1, srcp, dstp)

    ztp, tp0, g1 = pl.pallas_call(
        _tc2_body,
        grid=(GRID,),
        in_specs=[_parta(DH), _partb(DH), _COL, _COL, _full(1, DH),
                  _full(DH, DOUT), _full(1, DOUT)],
        out_specs=[_rows(DH), _rows(DOUT), _rows(DH)],
        out_shape=[_sd((NPAD, DH)), _sd((N, DOUT)), _sd((NPAD, DH))],
    )(p1, p1, ns, nd, b0, w1_W0, wb10)

    p2 = prop128(g1, srcp, dstp)

    emb0, h2 = pl.pallas_call(
        _tc3_body,
        grid=(GRID,),
        in_specs=[_rows(DH), _parta(DH), _partb(DH), _rows(DIN), _COL, _COL,
                  _full(DH, DH), _full(DIN, DH), _full(DH, DOUT),
                  _full(1, DOUT), _full(DH, DOUT)],
        out_specs=[_rows(DOUT), _rows(DOUT)],
        out_shape=[_sd((N, DOUT)), _sd((NPAD, DOUT))],
    )(ztp, p2, p2, xp, ns, nd, pai1_0, pai2_0, w2_W0, wb20, gcn_W1)

    p3 = prop64(h2, srcp, dstp)

    ztp2, tp1, g2 = pl.pallas_call(
        _tc4_body,
        grid=(GRID,),
        in_specs=[_parta(DOUT), _partb(DOUT), _COL, _COL, _full(1, DOUT),
                  _full(DOUT, DOUT), _full(1, DOUT)],
        out_specs=[_rows(DOUT), _rows(DOUT), _rows(DOUT)],
        out_shape=[_sd((NPAD, DOUT)), _sd((N, DOUT)), _sd((NPAD, DOUT))],
    )(p3, p3, ns, nd, b1, w1_W1, wb11)

    p4 = prop64(g2, srcp, dstp)

    (emb1,) = pl.pallas_call(
        _tc5_body,
        grid=(GRID,),
        in_specs=[_rows(DOUT), _parta(DOUT), _partb(DOUT), _rows(DIN), _COL,
                  _full(DOUT, DOUT), _full(DIN, DOUT), _full(DOUT, DOUT),
                  _full(1, DOUT)],
        out_specs=[_rows(DOUT)],
        out_shape=[_sd((N, DOUT))],
    )(ztp2, p4, p4, xp, nd, pai1_1, pai2_1, w2_W1, wb21)

    return (tp0, emb0, tp1, emb1)


# speedup vs baseline: 21.8398x; 1.0170x over previous
import functools

import jax
import jax.numpy as jnp
from jax import lax
from jax.experimental import pallas as pl
from jax.experimental.pallas import tpu as pltpu, tpu_sc as plsc

N = 10000
DIN = 128
DH = 128
DOUT = 64
TH1 = 0.1
TH2 = 1.0

NC = 2
NS = 16
NW = NC * NS
NPAD = 10240
ZR = NPAD // NS

CH = 128
EPW = 10240
NCHK = EPW // CH
EP = NW * EPW

R = 2048
GRID = NPAD // R

_f32 = jnp.float32


def _act(v):
    w1 = (2.0 * TH2 - TH1) / TH2
    w2 = w1 - 1.0
    return (w1 * (jax.nn.relu(v - TH1) - jax.nn.relu(-v - TH1))
            - w2 * (jax.nn.relu(v - TH2) - jax.nn.relu(-v - TH2)))


@functools.cache
def _mesh():
    return plsc.VectorSubcoreMesh(core_axis_name="c", subcore_axis_name="s",
                                  num_cores=NC, num_subcores=NS)


def _fill(buf, n, val, idx=()):
    def body(i, _):
        buf[idx + (pl.ds(i * 16, 16),)] = jnp.full((16,), val, _f32)
        return 0

    lax.fori_loop(0, n // 16, body, 0, unroll=8)


def _make_degree():
    K = 4
    PH = NCHK // K

    @functools.partial(
        pl.kernel,
        out_type=jax.ShapeDtypeStruct((4 * NPAD,), _f32),
        mesh=_mesh(),
        scratch_types=[
            pltpu.VMEM((2, K, CH), jnp.int32),
            pltpu.VMEM((2, K, CH), jnp.int32),
            pltpu.VMEM((CH,), _f32),
            pltpu.VMEM((ZR,), _f32),
            pltpu.VMEM_SHARED((NPAD,), _f32),
            pltpu.VMEM_SHARED((NPAD,), _f32),
            pltpu.SemaphoreType.DMA,
            pltpu.SemaphoreType.DMA,
            pltpu.SemaphoreType.DMA,
            pltpu.SemaphoreType.DMA,
        ],
    )
    def degree(earr, out_hbm, idxs, idxd, onesb, zb, acc_s, acc_d,
               isem0, isem1, ssem0, ssem1):
        c = lax.axis_index("c")
        s = lax.axis_index("s")
        w = c * NS + s
        isem = (isem0, isem1)
        ssem = (ssem0, ssem1)

        _fill(onesb, CH, 1.0)
        _fill(zb, ZR, 0.0)
        pltpu.sync_copy(zb, acc_s.at[pl.ds(s * ZR, ZR)])
        pltpu.sync_copy(zb, acc_d.at[pl.ds(s * ZR, ZR)])
        plsc.subcore_barrier()

        def fire_idx(p, x):
            pltpu.async_copy(earr.at[0, w, pl.ds(p * K, K)], idxs.at[x],
                             isem[x])
            pltpu.async_copy(earr.at[1, w, pl.ds(p * K, K)], idxd.at[x],
                             isem[x])

        def wait_idx(p, x):
            pltpu.make_async_copy(earr.at[0, w, pl.ds(p * K, K)], idxs.at[x],
                                  isem[x]).wait()
            pltpu.make_async_copy(earr.at[1, w, pl.ds(p * K, K)], idxd.at[x],
                                  isem[x]).wait()

        def drain_scatters(x):
            for k in range(K):
                pltpu.make_async_copy(onesb, acc_s.at[idxs.at[x, k]],
                                      ssem[x]).wait()
                pltpu.make_async_copy(onesb, acc_d.at[idxd.at[x, k]],
                                      ssem[x]).wait()

        def phase(p, x, first, last):
            y = 1 - x
            wait_idx(p, x)
            if first:
                pass
            else:
                drain_scatters(y)
            if last:
                pass
            else:
                @pl.when(p + 1 < PH)
                def _():
                    fire_idx(p + 1, y)
            for k in range(K):
                pltpu.async_copy(onesb, acc_s.at[idxs.at[x, k]], ssem[x],
                                 add=True)
                pltpu.async_copy(onesb, acc_d.at[idxd.at[x, k]], ssem[x],
                                 add=True)

        fire_idx(0, 0)

        def body(t, _):
            phase(2 * t, 0, False, False)
            phase(2 * t + 1, 1, False, False)
            return 0

        phase(0, 0, True, False)
        phase(1, 1, False, False)
        lax.fori_loop(1, PH // 2, body, 0)
        drain_scatters((PH - 1) % 2)
        plsc.subcore_barrier()
        pltpu.sync_copy(acc_s.at[pl.ds(s * ZR, ZR)],
                        out_hbm.at[pl.ds((2 * c) * NPAD + s * ZR, ZR)])
        pltpu.sync_copy(acc_d.at[pl.ds(s * ZR, ZR)],
                        out_hbm.at[pl.ds((2 * c + 1) * NPAD + s * ZR, ZR)])

    return degree


def _make_prop(d):
    K = 1 if d >= 128 else 4
    PH = NCHK // K
    NSL = 2 * K
    PRELOAD_SRC = d < 128

    src_slots = [] if PRELOAD_SRC else [pltpu.VMEM((CH,), jnp.int32)] * 2
    src_pre = [pltpu.VMEM((NCHK, CH), jnp.int32)] if PRELOAD_SRC else []

    @functools.partial(
        pl.kernel,
        out_type=jax.ShapeDtypeStruct((2 * NPAD, d), _f32),
        mesh=_mesh(),
        compiler_params=pltpu.CompilerParams(
            use_tc_tiling_on_sc=(d % 128 == 0)),
        scratch_types=(
            src_slots + src_pre
            + [pltpu.VMEM((NCHK, CH), jnp.int32)]
            + [pltpu.VMEM((CH, d), _f32)] * NSL
            + [pltpu.VMEM_SHARED((NPAD, d), _f32)]
            + [pltpu.SemaphoreType.DMA] * 5
        ),
    )
    def prop(h_hbm, earr, out_hbm, *sc):
        c = lax.axis_index("c")
        s = lax.axis_index("s")
        w = c * NS + s
        if PRELOAD_SRC:
            srcall = sc[0]
            nfix = 1
        else:
            ssl = sc[0:2]
            nfix = 2
        dstall = sc[nfix]
        rl = [sc[nfix + 1 + x * K:nfix + 1 + (x + 1) * K] for x in range(2)]
        acc = sc[nfix + 1 + NSL]
        isem = sc[nfix + 2 + NSL:nfix + 4 + NSL]
        gsem = sc[nfix + 4 + NSL]
        ssem = sc[nfix + 5 + NSL:nfix + 7 + NSL]

        pltpu.async_copy(earr.at[1, w], dstall, isem[0])
        if PRELOAD_SRC:
            pltpu.async_copy(earr.at[0, w], srcall, isem[1])
        zb = rl[0][0]

        def zrow(i, _):
            col = i % (d // 16)
            row = i // (d // 16)
            zb[row, pl.ds(col * 16, 16)] = jnp.zeros((16,), _f32)
            return 0

        lax.fori_loop(0, CH * d // 16, zrow, 0, unroll=8)
        for k in range(ZR // CH):
            pltpu.sync_copy(zb, acc.at[pl.ds(s * ZR + k * CH, CH)])
        pltpu.make_async_copy(earr.at[1, w], dstall, isem[0]).wait()
        if PRELOAD_SRC:
            pltpu.make_async_copy(earr.at[0, w], srcall, isem[1]).wait()
        plsc.subcore_barrier()

        def sidx(p, k, x):
            if PRELOAD_SRC:
                return srcall.at[p * K + k]
            return ssl[x]

        def drain_scatters(p, x):
            for k in range(K):
                pltpu.make_async_copy(rl[x][k], acc.at[dstall.at[p * K + k]],
                                      ssem[x]).wait()

        def phase(p, x, first):
            y = 1 - x
            if not PRELOAD_SRC:
                pltpu.make_async_copy(earr.at[0, w, p], ssl[x],
                                      isem[x]).wait()
            for k in range(K):
                pltpu.async_copy(h_hbm.at[sidx(p, k, x)], rl[x][k], gsem)
            if not first:
                drain_scatters(p - 1, y)
            if not PRELOAD_SRC:
                @pl.when(p + 1 < PH)
                def _():
                    pltpu.async_copy(earr.at[0, w, p + 1], ssl[y], isem[y])
            for k in range(K):
                pltpu.make_async_copy(h_hbm.at[sidx(p, k, x)], rl[x][k],
                                      gsem).wait()
            for k in range(K):
                pltpu.async_copy(rl[x][k], acc.at[dstall.at[p * K + k]],
                                 ssem[x], add=True)

        if not PRELOAD_SRC:
            pltpu.async_copy(earr.at[0, w, 0], ssl[0], isem[0])
        phase(0, 0, True)
        phase(1, 1, False)

        def body(t, _):
            phase(2 * t, 0, False)
            phase(2 * t + 1, 1, False)
            return 0

        lax.fori_loop(1, PH // 2, body, 0)
        drain_scatters(PH - 1, (PH - 1) % 2)
        plsc.subcore_barrier()
        pltpu.sync_copy(acc.at[pl.ds(s * ZR, ZR)],
                        out_hbm.at[pl.ds(c * NPAD + s * ZR, ZR)])

    return prop


def _cols_from_deg(deg):
    ds = deg[0:1] + deg[2:3]
    dd = deg[1:2] + deg[3:4]
    rs = lax.rsqrt(jnp.maximum(jnp.concatenate([ds, dd], axis=0), 1.0))
    eye2 = jnp.eye(2, dtype=_f32)
    cols = lax.dot_general(rs, eye2, (((0,), (0,)), ((), ())),
                           preferred_element_type=_f32)
    return cols[:, 0:1], cols[:, 1:2]


def _tc1_body(deg_ref, x_ref, w0_ref, ns_ref, nd_ref, h1_ref):
    ns, nd = _cols_from_deg(deg_ref[...])
    ns_ref[...] = ns
    nd_ref[...] = nd
    h1_ref[...] = jnp.dot(x_ref[...], w0_ref[...],
                          preferred_element_type=_f32) * ns


def _tc2_body(pa_ref, pb_ref, ns_ref, nd_ref, b0_ref, w10_ref, b10_ref,
              ztp_ref, tp0_ref, g1_ref):
    prop = (pa_ref[...] + pb_ref[...]) * nd_ref[...]
    z = jax.nn.relu(prop + b0_ref[...])
    ztp_ref[...] = z
    tp0_ref[...] = jnp.tanh(jnp.dot(z, w10_ref[...],
                                    preferred_element_type=_f32) + b10_ref[...])
    g1_ref[...] = z * ns_ref[...]


def _tc3_body(ztp_ref, pa_ref, pb_ref, x_ref, ns_ref, nd_ref, pai10_ref,
              pai20_ref, w20_ref, b20_ref, wg1_ref, emb0_ref, h2_ref):
    z = ztp_ref[...]
    propz = (pa_ref[...] + pb_ref[...]) * nd_ref[...]
    ze = (jnp.dot(z, pai10_ref[...], preferred_element_type=_f32)
          + jnp.dot(x_ref[...], pai20_ref[...], preferred_element_type=_f32)
          - z + propz)
    ze = _act(ze)
    emb0_ref[...] = jnp.tanh(jnp.dot(ze, w20_ref[...],
                                     preferred_element_type=_f32) + b20_ref[...])
    h2_ref[...] = jnp.dot(ze, wg1_ref[...],
                          preferred_element_type=_f32) * ns_ref[...]


def _tc4_body(pa_ref, pb_ref, ns_ref, nd_ref, b1_ref, w11_ref, b11_ref,
              ztp_ref, tp1_ref, g2_ref):
    prop = (pa_ref[...] + pb_ref[...]) * nd_ref[...]
    z = jax.nn.relu(prop + b1_ref[...])
    ztp_ref[...] = z
    tp1_ref[...] = jnp.tanh(jnp.dot(z, w11_ref[...],
                                    preferred_element_type=_f32) + b11_ref[...])
    g2_ref[...] = z * ns_ref[...]


def _tc5_body(ztp_ref, pa_ref, pb_ref, x_ref, nd_ref, pai11_ref, pai21_ref,
              w21_ref, b21_ref, emb1_ref):
    z = ztp_ref[...]
    propz = (pa_ref[...] + pb_ref[...]) * nd_ref[...]
    ze = (jnp.dot(z, pai11_ref[...], preferred_element_type=_f32)
          + jnp.dot(x_ref[...], pai21_ref[...], preferred_element_type=_f32)
          - z + propz)
    ze = _act(ze)
    emb1_ref[...] = jnp.tanh(jnp.dot(ze, w21_ref[...],
                                     preferred_element_type=_f32) + b21_ref[...])


def _rows(d):
    return pl.BlockSpec((R, d), lambda i: (i, 0))


def _parta(d):
    return pl.BlockSpec((R, d), lambda i: (i, 0))


def _partb(d):
    return pl.BlockSpec((R, d), lambda i: (i + GRID, 0))


def _full(a, b):
    return pl.BlockSpec((a, b), lambda i: (0, 0))


_COL = pl.BlockSpec((R, 1), lambda i: (i, 0))
_DEG = pl.BlockSpec((4, R), lambda i: (0, i))


def _sd(shape):
    return jax.ShapeDtypeStruct(shape, _f32)


def kernel(x, gcn_W0, gcn_b0, gcn_W1, gcn_b1, pai1_0, pai1_1, pai2_0, pai2_1,
           w1_W0, w1_b0, w1_W1, w1_b1, w2_W0, w2_b0, w2_W1, w2_b1, edge_index):
    e = edge_index.shape[1]
    eidx = edge_index.astype(jnp.int32)
    npadedge = EP - e
    pad_idx = jnp.broadcast_to(
        N + jnp.arange(NPAD - N, dtype=jnp.int32),
        (npadedge // (NPAD - N), NPAD - N)).reshape(npadedge)
    earr = jnp.concatenate(
        [eidx, jnp.broadcast_to(pad_idx, (2, npadedge))],
        axis=1).reshape(2, NW, NCHK, CH)

    xp = x
    b0 = gcn_b0.reshape(1, DH)
    b1 = gcn_b1.reshape(1, DOUT)
    wb10 = w1_b0.reshape(1, DOUT)
    wb11 = w1_b1.reshape(1, DOUT)
    wb20 = w2_b0.reshape(1, DOUT)
    wb21 = w2_b1.reshape(1, DOUT)

    deg = _make_degree()(earr).reshape(4, NPAD)

    ns, nd, h1 = pl.pallas_call(
        _tc1_body,
        grid=(GRID,),
        in_specs=[_DEG, _rows(DIN), _full(DIN, DH)],
        out_specs=[_COL, _COL, _rows(DH)],
        out_shape=[_sd((NPAD, 1)), _sd((NPAD, 1)), _sd((NPAD, DH))],
    )(deg, xp, gcn_W0)

    prop128 = _make_prop(DH)
    prop64 = _make_prop(DOUT)

    p1 = prop128(h1, earr)

    ztp, tp0, g1 = pl.pallas_call(
        _tc2_body,
        grid=(GRID,),
        in_specs=[_parta(DH), _partb(DH), _COL, _COL, _full(1, DH),
                  _full(DH, DOUT), _full(1, DOUT)],
        out_specs=[_rows(DH), _rows(DOUT), _rows(DH)],
        out_shape=[_sd((NPAD, DH)), _sd((N, DOUT)), _sd((NPAD, DH))],
    )(p1, p1, ns, nd, b0, w1_W0, wb10)

    p2 = prop128(g1, earr)

    emb0, h2 = pl.pallas_call(
        _tc3_body,
        grid=(GRID,),
        in_specs=[_rows(DH), _parta(DH), _partb(DH), _rows(DIN), _COL, _COL,
                  _full(DH, DH), _full(DIN, DH), _full(DH, DOUT),
                  _full(1, DOUT), _full(DH, DOUT)],
        out_specs=[_rows(DOUT), _rows(DOUT)],
        out_shape=[_sd((N, DOUT)), _sd((NPAD, DOUT))],
    )(ztp, p2, p2, xp, ns, nd, pai1_0, pai2_0, w2_W0, wb20, gcn_W1)

    p3 = prop64(h2, earr)

    ztp2, tp1, g2 = pl.pallas_call(
        _tc4_body,
        grid=(GRID,),
        in_specs=[_parta(DOUT), _partb(DOUT), _COL, _COL, _full(1, DOUT),
                  _full(DOUT, DOUT), _full(1, DOUT)],
        out_specs=[_rows(DOUT), _rows(DOUT), _rows(DOUT)],
        out_shape=[_sd((NPAD, DOUT)), _sd((N, DOUT)), _sd((NPAD, DOUT))],
    )(p3, p3, ns, nd, b1, w1_W1, wb11)

    p4 = prop64(g2, earr)

    (emb1,) = pl.pallas_call(
        _tc5_body,
        grid=(GRID,),
        in_specs=[_rows(DOUT), _parta(DOUT), _partb(DOUT), _rows(DIN), _COL,
                  _full(DOUT, DOUT), _full(DIN, DOUT), _full(DOUT, DOUT),
                  _full(1, DOUT)],
        out_specs=[_rows(DOUT)],
        out_shape=[_sd((N, DOUT))],
    )(ztp2, p4, p4, xp, nd, pai1_1, pai2_1, w2_W1, wb21)

    return (tp0, emb0, tp1, emb1)
